# baseline (speedup 1.0000x reference)
import functools

import jax
import jax.numpy as jnp
from jax import lax
from jax.experimental import pallas as pl
from jax.experimental.pallas import tpu as pltpu


NUM_CLASSES = 64


def _fused_kernel(s_ref, lab_ref, q_ref, outs_ref, outq_ref,
                  pn_s, pb2_s, deg_s, qbf_s, psum_s, mean_s, *, k, qb, nq):
    i = pl.program_id(0)

    @pl.when(i == 0)
    def _():
        labels = lab_ref[...]
        classes = lax.broadcasted_iota(
            jnp.int32, (NUM_CLASSES, labels.shape[1]), 0)
        onehot = (labels == classes).astype(jnp.float32)
        sums = jnp.dot(onehot, s_ref[...], preferred_element_type=jnp.float32)
        counts = jnp.sum(onehot, axis=1, keepdims=True)
        protos = sums / jnp.maximum(counts, 1.0)
        psum_s[...] = jnp.sum(protos, axis=0, keepdims=True)
        norm = jnp.sqrt(jnp.sum(protos * protos, axis=1, keepdims=True))
        pn = protos / jnp.maximum(norm, 1e-12)
        pn_s[...] = pn
        pb2_s[...] = jnp.sum(pn * pn, axis=1)[None, :]

    @pl.when(i < nq)
    def _():
        q = q_ref[...]
        qbf_s[pl.ds(i, 1), :, :] = q.astype(jnp.bfloat16)[None]
        q2 = jnp.sum(q * q, axis=1, keepdims=True)
        inv_norm = lax.rsqrt(jnp.maximum(q2, 1e-24))
        cos = lax.dot_general(q, pn_s[...], (((1,), (1,)), ((), ()))) * inv_norm
        d2 = 1.0 + pb2_s[...] - 2.0 * cos
        deg = jnp.sum(jnp.sqrt(jnp.maximum(d2, 1e-12)), axis=1)
        rows = qb // 128
        deg_s[pl.ds(i * rows, rows), :] = deg.reshape(rows, 128)

    @pl.when(i == nq)
    def _():
        bits = lax.bitcast_convert_type(deg_s[...], jnp.int32)

        def body(_, carry):
            lo, hi = carry
            mid = lo + (hi - lo + 1) // 2
            cnt = jnp.sum((bits >= mid).astype(jnp.int32))
            ok = cnt >= k
            return jnp.where(ok, mid, lo), jnp.where(ok, hi, mid - 1)

        lo, _ = lax.fori_loop(0, 31, body,
                              (jnp.int32(0), jnp.int32(0x7F800000)))
        gt = bits > lo
        eq = bits == lo
        m = k - jnp.sum(gt.astype(jnp.int32))

        n = bits.shape[0]
        i_idx = lax.broadcasted_iota(jnp.int32, (n, n), 0)
        j_idx = lax.broadcasted_iota(jnp.int32, (n, n), 1)
        lower_strict = (j_idx < i_idx).astype(jnp.float32)
        upper_strict = (i_idx < j_idx).astype(jnp.float32)
        eqf = eq.astype(jnp.float32)
        row_tot = jnp.sum(eqf, axis=1, keepdims=True)
        row_excl = jnp.dot(lower_strict, row_tot,
                           preferred_element_type=jnp.float32)
        col_excl = jnp.dot(eqf, upper_strict,
                           preferred_element_type=jnp.float32)
        prefix = (row_excl + col_excl).astype(jnp.int32)
        mask = jnp.where(gt | (eq & (prefix < m)), 1.0, 0.0
                         ).astype(jnp.bfloat16)

        rpb = qb // 128
        acc = jnp.zeros((1, qbf_s.shape[2]), jnp.float32)
        for r in range(n):
            qrows = qbf_s[r // rpb, pl.ds((r % rpb) * 128, 128), :]
            acc = acc + jnp.dot(mask[r:r + 1, :], qrows,
                                preferred_element_type=jnp.float32)
        mean = (acc + psum_s[...]) * (1.0 / (NUM_CLASSES + k))
        mean_s[...] = mean
        outs_ref[...] = s_ref[...] - mean

    @pl.when(i > nq)
    def _():
        b = i - nq - 1
        rows = qbf_s[pl.ds(b, 1), :, :].astype(jnp.float32)
        outq_ref[...] = rows[0] - mean_s[...]


def kernel(feat_s, feat_q, support_labels):
    S, D = feat_s.shape
    Q = feat_q.shape[0]
    C = NUM_CLASSES
    k = min(Q, max(S, Q // 4))
    R = 128

    labels = support_labels.astype(jnp.int32).reshape(1, S)

    QB = 512
    nq = Q // QB
    out_s, out_q = pl.pallas_call(
        functools.partial(_fused_kernel, k=k, qb=QB, nq=nq),
        grid=(2 * nq + 1,),
        in_specs=[
            pl.BlockSpec((S, D), lambda i: (0, 0)),
            pl.BlockSpec((1, S), lambda i: (0, 0)),
            pl.BlockSpec((QB, D), lambda i: (jnp.minimum(i, nq - 1), 0)),
        ],
        out_specs=(
            pl.BlockSpec((S, D), lambda i: (0, 0)),
            pl.BlockSpec((QB, D),
                         lambda i: (jnp.maximum(i - nq - 1, 0), 0)),
        ),
        out_shape=(
            jax.ShapeDtypeStruct((S, D), jnp.float32),
            jax.ShapeDtypeStruct((Q, D), jnp.float32),
        ),
        scratch_shapes=[
            pltpu.VMEM((C, D), jnp.float32),
            pltpu.VMEM((1, C), jnp.float32),
            pltpu.VMEM((R, R), jnp.float32),
            pltpu.VMEM((nq, QB, D), jnp.bfloat16),
            pltpu.VMEM((1, D), jnp.float32),
            pltpu.VMEM((1, D), jnp.float32),
        ],
    )(feat_s, labels, feat_q)

    return out_s, out_q


# speedup vs baseline: 1.2253x; 1.2253x over previous
import functools

import jax
import jax.numpy as jnp
from jax import lax
from jax.experimental import pallas as pl
from jax.experimental.pallas import tpu as pltpu


NUM_CLASSES = 64


def _fused_kernel(s_ref, lab_ref, q_ref, outs_ref, outq_ref,
                  pn_s, pb2_s, deg_s, qbf_s, psum_s, mean_s, *, k, qb, nq):
    i = pl.program_id(0)

    @pl.when(i == 0)
    def _():
        labels = lab_ref[...]
        classes = lax.broadcasted_iota(
            jnp.int32, (NUM_CLASSES, labels.shape[1]), 0)
        onehot = (labels == classes).astype(jnp.float32)
        sums = jnp.dot(onehot, s_ref[...], preferred_element_type=jnp.float32)
        counts = jnp.sum(onehot, axis=1, keepdims=True)
        protos = sums / jnp.maximum(counts, 1.0)
        psum_s[...] = jnp.sum(protos, axis=0, keepdims=True)
        norm = jnp.sqrt(jnp.sum(protos * protos, axis=1, keepdims=True))
        pn = protos / jnp.maximum(norm, 1e-12)
        pn_s[...] = pn
        pb2_s[...] = jnp.sum(pn * pn, axis=1)[None, :]

    @pl.when(i < nq)
    def _():
        q = q_ref[...]
        qbf_s[pl.ds(i, 1), :, :] = q.astype(jnp.bfloat16)[None]
        q2 = jnp.sum(q * q, axis=1, keepdims=True)
        inv_norm = lax.rsqrt(jnp.maximum(q2, 1e-24))
        cos = lax.dot_general(q, pn_s[...], (((1,), (1,)), ((), ()))) * inv_norm
        d2 = 1.0 + pb2_s[...] - 2.0 * cos
        deg = jnp.sum(jnp.sqrt(jnp.maximum(d2, 1e-12)), axis=1)
        rows = qb // 128
        deg_s[pl.ds(i * rows, rows), :] = deg.reshape(rows, 128)

    @pl.when(i == nq)
    def _():
        bits = lax.bitcast_convert_type(deg_s[...], jnp.int32)

        def body(_, carry):
            lo, hi = carry
            mid = lo + (hi - lo + 1) // 2
            cnt = jnp.sum((bits >= mid).astype(jnp.int32))
            ok = cnt >= k
            return jnp.where(ok, mid, lo), jnp.where(ok, hi, mid - 1)

        lo, _ = lax.fori_loop(0, 31, body,
                              (jnp.int32(0), jnp.int32(0x7F800000)))
        gt = bits > lo
        eq = bits == lo
        m = k - jnp.sum(gt.astype(jnp.int32))

        n = bits.shape[0]
        i_idx = lax.broadcasted_iota(jnp.int32, (n, n), 0)
        j_idx = lax.broadcasted_iota(jnp.int32, (n, n), 1)
        lower_strict = (j_idx < i_idx).astype(jnp.float32)
        upper_strict = (i_idx < j_idx).astype(jnp.float32)
        eqf = eq.astype(jnp.float32)
        row_tot = jnp.sum(eqf, axis=1, keepdims=True)
        row_excl = jnp.dot(lower_strict, row_tot,
                           preferred_element_type=jnp.float32)
        col_excl = jnp.dot(eqf, upper_strict,
                           preferred_element_type=jnp.float32)
        prefix = (row_excl + col_excl).astype(jnp.int32)
        mask = jnp.where(gt | (eq & (prefix < m)), 1.0, 0.0
                         ).astype(jnp.bfloat16)

        rpb = qb // 128
        acc = jnp.zeros((1, qbf_s.shape[2]), jnp.float32)
        for r in range(n):
            qrows = qbf_s[r // rpb, pl.ds((r % rpb) * 128, 128), :]
            acc = acc + jnp.dot(mask[r:r + 1, :], qrows,
                                preferred_element_type=jnp.float32)
        mean = (acc + psum_s[...]) * (1.0 / (NUM_CLASSES + k))
        mean_s[...] = mean
        outs_ref[...] = s_ref[...] - mean

    @pl.when(i > nq)
    def _():
        b = i - nq - 1
        rows = qbf_s[pl.ds(b, 1), :, :].astype(jnp.float32)
        outq_ref[...] = rows[0] - mean_s[...]


def kernel(feat_s, feat_q, support_labels):
    S, D = feat_s.shape
    Q = feat_q.shape[0]
    C = NUM_CLASSES
    k = min(Q, max(S, Q // 4))
    R = 128

    labels = support_labels.astype(jnp.int32).reshape(1, S)

    QB = 1024
    nq = Q // QB
    out_s, out_q = pl.pallas_call(
        functools.partial(_fused_kernel, k=k, qb=QB, nq=nq),
        grid=(2 * nq + 1,),
        in_specs=[
            pl.BlockSpec((S, D), lambda i: (0, 0)),
            pl.BlockSpec((1, S), lambda i: (0, 0)),
            pl.BlockSpec((QB, D), lambda i: (jnp.minimum(i, nq - 1), 0)),
        ],
        out_specs=(
            pl.BlockSpec((S, D), lambda i: (0, 0)),
            pl.BlockSpec((QB, D),
                         lambda i: (jnp.maximum(i - nq - 1, 0), 0)),
        ),
        out_shape=(
            jax.ShapeDtypeStruct((S, D), jnp.float32),
            jax.ShapeDtypeStruct((Q, D), jnp.float32),
        ),
        scratch_shapes=[
            pltpu.VMEM((C, D), jnp.float32),
            pltpu.VMEM((1, C), jnp.float32),
            pltpu.VMEM((R, R), jnp.float32),
            pltpu.VMEM((nq, QB, D), jnp.bfloat16),
            pltpu.VMEM((1, D), jnp.float32),
            pltpu.VMEM((1, D), jnp.float32),
        ],
    )(feat_s, labels, feat_q)

    return out_s, out_q
